# baseline (speedup 1.0000x reference)
import functools

import jax
import jax.numpy as jnp
from jax.experimental import pallas as pl
from jax.experimental.pallas import tpu as pltpu

D = 1024
E = 8
K = 2
H = 4096
M = 512
NB = 23
NRP = NB * M
HB = 512
NH = H // HB


def _hidden_kernel(be_ref, na_ref, xg_ref, gate_ref, up_ref, hid_ref):
    b = pl.program_id(1)

    @pl.when(b < na_ref[0])
    def _():
        xg = xg_ref[...]
        g = jax.lax.dot_general(
            xg, gate_ref[0].astype(jnp.bfloat16), (((1,), (1,)), ((), ())),
            preferred_element_type=jnp.float32)
        u = jax.lax.dot_general(
            xg, up_ref[0].astype(jnp.bfloat16), (((1,), (1,)), ((), ())),
            preferred_element_type=jnp.float32)
        hid_ref[...] = (g * jax.nn.sigmoid(g) * u).astype(jnp.bfloat16)


def _down_kernel(be_ref, na_ref, hid_ref, down_ref, o_ref, acc_ref):
    b = pl.program_id(0)
    h = pl.program_id(1)

    @pl.when(b < na_ref[0])
    def _():
        yp = jax.lax.dot_general(
            hid_ref[...], down_ref[0].astype(jnp.bfloat16),
            (((1,), (1,)), ((), ())), preferred_element_type=jnp.float32)

        @pl.when(h == 0)
        def _():
            acc_ref[...] = yp

        @pl.when(h > 0)
        def _():
            acc_ref[...] += yp

        @pl.when(h == NH - 1)
        def _():
            o_ref[...] = acc_ref[...]


def _grouped_ffn(block_expert, nact, xg, gate_W, up_W, down_W):
    def _row_ix_a(h, b, be, na):
        return (jnp.where(b < na[0], b, jnp.minimum(na[0], NB - 1)), 0)

    def _gu_ix(h, b, be, na):
        return (be[b], jnp.where(b < na[0], h, 0), 0)

    def _hid_out_ix(h, b, be, na):
        return (jnp.where(b < na[0], b, jnp.minimum(na[0], NB - 1)), h)

    grid_a = pltpu.PrefetchScalarGridSpec(
        num_scalar_prefetch=2,
        grid=(NH, NB),
        in_specs=[
            pl.BlockSpec((M, D), _row_ix_a),
            pl.BlockSpec((1, HB, D), _gu_ix),
            pl.BlockSpec((1, HB, D), _gu_ix),
        ],
        out_specs=pl.BlockSpec((M, HB), _hid_out_ix),
    )
    hid = pl.pallas_call(
        _hidden_kernel,
        grid_spec=grid_a,
        out_shape=jax.ShapeDtypeStruct((NRP, H), jnp.bfloat16),
    )(block_expert, nact, xg, gate_W, up_W)

    def _hid_in_ix(b, h, be, na):
        valid = b < na[0]
        return (jnp.where(valid, b, jnp.minimum(na[0], NB - 1)),
                jnp.where(valid, h, 0))

    def _dn_ix(b, h, be, na):
        return (be[b], 0, jnp.where(b < na[0], h, 0))

    def _y_out_ix(b, h, be, na):
        return (jnp.where(b < na[0], b, jnp.minimum(na[0], NB - 1)), 0)

    grid_b = pltpu.PrefetchScalarGridSpec(
        num_scalar_prefetch=2,
        grid=(NB, NH),
        in_specs=[
            pl.BlockSpec((M, HB), _hid_in_ix),
            pl.BlockSpec((1, D, HB), _dn_ix),
        ],
        out_specs=pl.BlockSpec((M, D), _y_out_ix),
        scratch_shapes=[pltpu.VMEM((M, D), jnp.float32)],
    )
    return pl.pallas_call(
        _down_kernel,
        grid_spec=grid_b,
        out_shape=jax.ShapeDtypeStruct((NRP, D), jnp.float32),
    )(block_expert, nact, hid, down_W)


def kernel(x, router_W, gate_W, up_W, down_W):
    Bx, Tx, Dx = x.shape
    N = Bx * Tx
    xf = x.reshape(N, Dx)

    logits = (xf @ router_W.T).astype(jnp.float32)
    top_logits, top_idx = jax.lax.top_k(logits, K)
    top_w = jax.nn.softmax(top_logits, axis=-1).astype(x.dtype)

    pairs_e = top_idx.T.reshape(-1)
    onehot = (pairs_e[:, None] == jnp.arange(E)[None, :]).astype(jnp.int32)
    ranks = jnp.cumsum(onehot, axis=0) - onehot
    rank = jnp.take_along_axis(ranks, pairs_e[:, None], axis=1)[:, 0]
    counts = jnp.sum(onehot, axis=0)
    blocks = (counts + M - 1) // M
    cum_blocks = jnp.cumsum(blocks)
    padded_start = (cum_blocks - blocks) * M
    dest = padded_start[pairs_e] + rank

    pairs_t = jnp.tile(jnp.arange(N, dtype=jnp.int32), K)
    token_of_row = jnp.zeros((NRP,), jnp.int32).at[dest].set(pairs_t)
    block_expert = jnp.searchsorted(
        cum_blocks, jnp.arange(NB, dtype=jnp.int32), side='right')
    block_expert = jnp.minimum(block_expert, E - 1).astype(jnp.int32)

    nact = cum_blocks[-1:].astype(jnp.int32)
    xg = xf.astype(jnp.bfloat16)[token_of_row]
    y = _grouped_ffn(block_expert, nact, xg, gate_W, up_W, down_W)

    r1 = dest[:N]
    r2 = dest[N:]
    out = top_w[:, 0:1] * y[r1] + top_w[:, 1:2] * y[r2]
    return out.reshape(Bx, Tx, Dx).astype(x.dtype)


# speedup vs baseline: 1.2236x; 1.2236x over previous
import functools

import jax
import jax.numpy as jnp
from jax.experimental import pallas as pl
from jax.experimental.pallas import tpu as pltpu

D = 1024
E = 8
K = 2
H = 4096
M = 1024
NB = 15
NRP = NB * M
HB = 512
NH = H // HB


def _ffn_kernel(be_ref, na_ref, xg_ref, gate_ref, up_ref, down_ref, o_ref,
                acc_ref):
    b = pl.program_id(0)
    h = pl.program_id(1)

    @pl.when(b < na_ref[0])
    def _():
        xg = xg_ref[...]
        g = jax.lax.dot_general(
            xg, gate_ref[0].astype(jnp.bfloat16), (((1,), (1,)), ((), ())),
            preferred_element_type=jnp.float32)
        u = jax.lax.dot_general(
            xg, up_ref[0].astype(jnp.bfloat16), (((1,), (1,)), ((), ())),
            preferred_element_type=jnp.float32)
        hact = (g * jax.nn.sigmoid(g) * u).astype(jnp.bfloat16)
        yp = jax.lax.dot_general(
            hact, down_ref[0].astype(jnp.bfloat16), (((1,), (1,)), ((), ())),
            preferred_element_type=jnp.float32)

        @pl.when(h == 0)
        def _():
            acc_ref[...] = yp

        @pl.when(h > 0)
        def _():
            acc_ref[...] += yp

        @pl.when(h == NH - 1)
        def _():
            o_ref[...] = acc_ref[...]


def _grouped_ffn(block_expert, nact, xg, gate_W, up_W, down_W):
    def _row_ix(b, h, be, na):
        return (jnp.where(b < na[0], b, jnp.minimum(na[0], NB - 1)), 0)

    def _gu_ix(b, h, be, na):
        return (be[b], jnp.where(b < na[0], h, 0), 0)

    def _dn_ix(b, h, be, na):
        return (be[b], 0, jnp.where(b < na[0], h, 0))

    grid_spec = pltpu.PrefetchScalarGridSpec(
        num_scalar_prefetch=2,
        grid=(NB, NH),
        in_specs=[
            pl.BlockSpec((M, D), _row_ix),
            pl.BlockSpec((1, HB, D), _gu_ix),
            pl.BlockSpec((1, HB, D), _gu_ix),
            pl.BlockSpec((1, D, HB), _dn_ix),
        ],
        out_specs=pl.BlockSpec((M, D), _row_ix),
        scratch_shapes=[pltpu.VMEM((M, D), jnp.float32)],
    )
    return pl.pallas_call(
        _ffn_kernel,
        grid_spec=grid_spec,
        out_shape=jax.ShapeDtypeStruct((NRP, D), jnp.float32),
    )(block_expert, nact, xg, gate_W, up_W, down_W)


def kernel(x, router_W, gate_W, up_W, down_W):
    Bx, Tx, Dx = x.shape
    N = Bx * Tx
    xf = x.reshape(N, Dx)

    logits = (xf @ router_W.T).astype(jnp.float32)
    top_logits, top_idx = jax.lax.top_k(logits, K)
    top_w = jax.nn.softmax(top_logits, axis=-1).astype(x.dtype)

    pairs_e = top_idx.T.reshape(-1)
    onehot = (pairs_e[:, None] == jnp.arange(E)[None, :]).astype(jnp.int32)
    ranks = jnp.cumsum(onehot, axis=0) - onehot
    rank = jnp.take_along_axis(ranks, pairs_e[:, None], axis=1)[:, 0]
    counts = jnp.sum(onehot, axis=0)
    blocks = (counts + M - 1) // M
    cum_blocks = jnp.cumsum(blocks)
    padded_start = (cum_blocks - blocks) * M
    dest = padded_start[pairs_e] + rank

    pairs_t = jnp.tile(jnp.arange(N, dtype=jnp.int32), K)
    token_of_row = jnp.zeros((NRP,), jnp.int32).at[dest].set(pairs_t)
    block_expert = jnp.searchsorted(
        cum_blocks, jnp.arange(NB, dtype=jnp.int32), side='right')
    block_expert = jnp.minimum(block_expert, E - 1).astype(jnp.int32)

    nact = cum_blocks[-1:].astype(jnp.int32)
    xg = xf.astype(jnp.bfloat16)[token_of_row]
    y = _grouped_ffn(block_expert, nact, xg, gate_W, up_W, down_W)

    r1 = dest[:N]
    r2 = dest[N:]
    out = top_w[:, 0:1] * y[r1] + top_w[:, 1:2] * y[r2]
    return out.reshape(Bx, Tx, Dx).astype(x.dtype)


# speedup vs baseline: 1.2592x; 1.0291x over previous
import functools

import jax
import jax.numpy as jnp
from jax.experimental import pallas as pl
from jax.experimental.pallas import tpu as pltpu

D = 1024
E = 8
K = 2
H = 4096
M = 512
NB = 23
NRP = NB * M
HB = 512
NH = H // HB


def _ffn_kernel(be_ref, na_ref, xg_ref, gate_ref, up_ref, down_ref, o_ref,
                acc_ref):
    b = pl.program_id(0)
    h = pl.program_id(1)

    @pl.when(b < na_ref[0])
    def _():
        xg = xg_ref[...]
        g = jax.lax.dot_general(
            xg, gate_ref[0].astype(jnp.bfloat16), (((1,), (1,)), ((), ())),
            preferred_element_type=jnp.float32)
        u = jax.lax.dot_general(
            xg, up_ref[0].astype(jnp.bfloat16), (((1,), (1,)), ((), ())),
            preferred_element_type=jnp.float32)
        hact = (g * jax.nn.sigmoid(g) * u).astype(jnp.bfloat16)
        yp = jax.lax.dot_general(
            hact, down_ref[0].astype(jnp.bfloat16), (((1,), (1,)), ((), ())),
            preferred_element_type=jnp.float32)

        @pl.when(h == 0)
        def _():
            acc_ref[...] = yp

        @pl.when(h > 0)
        def _():
            acc_ref[...] += yp

        @pl.when(h == NH - 1)
        def _():
            o_ref[...] = acc_ref[...]


def _grouped_ffn(block_expert, nact, xg, gate_W, up_W, down_W):
    def _row_ix(b, h, be, na):
        return (jnp.where(b < na[0], b, jnp.minimum(na[0], NB - 1)), 0)

    def _gu_ix(b, h, be, na):
        return (be[b], jnp.where(b < na[0], h, 0), 0)

    def _dn_ix(b, h, be, na):
        return (be[b], 0, jnp.where(b < na[0], h, 0))

    grid_spec = pltpu.PrefetchScalarGridSpec(
        num_scalar_prefetch=2,
        grid=(NB, NH),
        in_specs=[
            pl.BlockSpec((M, D), _row_ix),
            pl.BlockSpec((1, HB, D), _gu_ix),
            pl.BlockSpec((1, HB, D), _gu_ix),
            pl.BlockSpec((1, D, HB), _dn_ix),
        ],
        out_specs=pl.BlockSpec((M, D), _row_ix),
        scratch_shapes=[pltpu.VMEM((M, D), jnp.float32)],
    )
    return pl.pallas_call(
        _ffn_kernel,
        grid_spec=grid_spec,
        out_shape=jax.ShapeDtypeStruct((NRP, D), jnp.float32),
    )(block_expert, nact, xg, gate_W, up_W, down_W)


def kernel(x, router_W, gate_W, up_W, down_W):
    Bx, Tx, Dx = x.shape
    N = Bx * Tx
    xf = x.reshape(N, Dx)

    logits = (xf @ router_W.T).astype(jnp.float32)
    idx1 = jnp.argmax(logits, axis=-1).astype(jnp.int32)
    m1 = jnp.max(logits, axis=-1)
    eids = jnp.arange(E, dtype=jnp.int32)
    masked = jnp.where(idx1[:, None] == eids[None, :], -jnp.inf, logits)
    idx2 = jnp.argmax(masked, axis=-1).astype(jnp.int32)
    m2 = jnp.max(masked, axis=-1)
    e2 = jnp.exp(m2 - m1)
    w1 = 1.0 / (1.0 + e2)
    w2 = e2 * w1
    top_w = jnp.stack([w1, w2], axis=1).astype(x.dtype)

    pairs_e = jnp.concatenate([idx1, idx2])
    onehot = (pairs_e[:, None] == eids[None, :]).astype(jnp.int32)
    ranks = jnp.cumsum(onehot, axis=0) - onehot
    rank = jnp.sum(ranks * onehot, axis=1)
    counts = onehot.sum(axis=0)
    blocks = (counts + M - 1) // M
    cum_blocks = jnp.cumsum(blocks)
    padded_start = (cum_blocks - blocks) * M
    dest = padded_start[pairs_e] + rank

    pairs_t = jnp.tile(jnp.arange(N, dtype=jnp.int32), K)
    token_of_row = jnp.zeros((NRP,), jnp.int32).at[dest].set(pairs_t)
    block_expert = jnp.searchsorted(
        cum_blocks, jnp.arange(NB, dtype=jnp.int32), side='right')
    block_expert = jnp.minimum(block_expert, E - 1).astype(jnp.int32)

    nact = cum_blocks[-1:].astype(jnp.int32)
    xg = xf.astype(jnp.bfloat16)[token_of_row]
    y = _grouped_ffn(block_expert, nact, xg, gate_W, up_W, down_W)

    r1 = dest[:N]
    r2 = dest[N:]
    out = top_w[:, 0:1] * y[r1] + top_w[:, 1:2] * y[r2]
    return out.reshape(Bx, Tx, Dx).astype(x.dtype)
